# Initial kernel scaffold; baseline (speedup 1.0000x reference)
import jax
import jax.numpy as jnp
from jax.experimental import pallas as pl


def kernel(feat_x, feat_y):
    raise NotImplementedError("write your pallas kernel here")


# speedup vs baseline: 6.1452x; 6.1452x over previous
import functools

import jax
import jax.numpy as jnp
from jax import lax
from jax.experimental import pallas as pl
from jax.experimental.pallas import tpu as pltpu
from jax.experimental.pallas import tpu_sc as plsc

_TAU = 0.2
_K = 15
_CHUNK = 2048
_GRP = 128
_GPC = _CHUNK // _GRP
_NEG = -1.0e30


def _tc_body(ny, fx_ref, y_ref, sim_ref, g_ref):
    c = pl.program_id(1)
    fx = fx_ref[0]
    xn = jnp.sqrt(jnp.sum(fx * fx, axis=1, keepdims=True))
    fxn = fx / jnp.maximum(xn, 1e-12)
    y = y_ref[0]
    yn = jnp.sqrt(jnp.sum(y * y, axis=1, keepdims=True))
    fyn = y / jnp.maximum(yn, 1e-12)
    sim = lax.dot_general(
        fxn, fyn, (((1,), (1,)), ((), ())),
        preferred_element_type=jnp.float32,
    ) / _TAU
    col = c * _CHUNK + lax.broadcasted_iota(jnp.int32, sim.shape, 1)
    sim = jnp.where(col < ny, sim, _NEG)
    sim_ref[0] = sim
    parts = [
        jnp.max(sim[:, g * _GRP:(g + 1) * _GRP], axis=1, keepdims=True)
        for g in range(_GPC)
    ]
    g_ref[0, 0] = jnp.concatenate(parts, axis=1)


def _make_sc_kernel(nrows, nchunks, ngroups):
    rpw = nrows // 32
    mesh = plsc.VectorSubcoreMesh(core_axis_name="c", subcore_axis_name="s")
    imax = jnp.int32(2**31 - 1)

    @functools.partial(
        pl.kernel,
        out_type=[
            jax.ShapeDtypeStruct((nrows * 16,), jnp.float32),
            jax.ShapeDtypeStruct((nrows * 16,), jnp.int32),
        ],
        mesh=mesh,
        scratch_types=[
            pltpu.VMEM((nchunks * rpw * _GPC,), jnp.float32),
            pltpu.VMEM((16, _GRP), jnp.float32),
            pltpu.VMEM((rpw * 16,), jnp.float32),
            pltpu.VMEM((rpw * 16,), jnp.int32),
            pltpu.SemaphoreType.DMA,
            pltpu.SemaphoreType.DMA,
        ],
        compiler_params=pltpu.CompilerParams(needs_layout_passes=False),
    )
    def sc_kernel(sim_hbm, g_hbm, outv_hbm, outc_hbm,
                  g_v, slab_v, ov_v, oc_v, sem_g, sem_s):
        wid = lax.axis_index("s") * 2 + lax.axis_index("c")
        row0 = wid * rpw
        b = row0 // 1024
        r0 = row0 % 1024
        lane = lax.iota(jnp.int32, 16)

        gsz = rpw * _GPC
        nxg = 1024 * _GPC
        copies = [
            pltpu.async_copy(
                g_hbm.at[pl.ds((b * nchunks + j) * nxg + r0 * _GPC, gsz)],
                g_v.at[pl.ds(j * gsz, gsz)], sem_g)
            for j in range(nchunks)
        ]
        for cp in copies:
            cp.wait()

        def merge16(rk, rv, nk, nv):
            nk, nv = plsc.sort_key_val(nk, nv, descending=True)
            nk = lax.rev(nk, (0,))
            nv = lax.rev(nv, (0,))
            m = rk >= nk
            mk = jnp.where(m, rk, nk)
            mv = jnp.where(m, rv, nv)
            ok, ov = plsc.sort_key_val(mk, mv, descending=True)
            return ok, ov

        def row_body(r, carry):
            def ga(j, ac):
                rk, rv = ac
                gvals = g_v[pl.ds(j * gsz + r * _GPC, 16)]
                gids = j * _GPC + lane
                return merge16(rk, rv, gvals, gids)

            rk, rv = lax.fori_loop(
                0, nchunks, ga,
                (jnp.full((16,), _NEG, jnp.float32), jnp.zeros((16,), jnp.int32)),
            )

            ids = (row0 + r) * ngroups + rv
            pltpu.async_copy(sim_hbm.at[ids], slab_v, sem_s).wait()

            def gs(s, sc):
                gbase = jnp.sum(jnp.where(lane == s, rv, 0)) * _GRP

                svec = jnp.full((16,), 0, jnp.int32) + s

                def gv(v, vc):
                    ck, cv = vc
                    off = v * 16 + lane
                    vals = plsc.load_gather(slab_v, [svec, off])
                    cols = gbase + off
                    return merge16(ck, cv, vals, cols)

                return lax.fori_loop(0, 8, gv, sc)

            ck, cv = lax.fori_loop(
                0, 16, gs,
                (jnp.full((16,), _NEG, jnp.float32), jnp.zeros((16,), jnp.int32)),
            )

            vmax = jnp.max(ck)
            e = jnp.where(lane < _K, jnp.exp(ck - vmax), 0.0)
            sm = e / jnp.sum(e)

            keys = jnp.where(lane < _K, cv, imax)
            sk, sv = plsc.sort_key_val(keys, sm, descending=False)
            ov_v[pl.ds(r * 16, 16)] = sv
            oc_v[pl.ds(r * 16, 16)] = sk
            return carry

        lax.fori_loop(0, rpw, row_body, jnp.int32(0))

        pltpu.sync_copy(ov_v, outv_hbm.at[pl.ds(row0 * 16, rpw * 16)])
        pltpu.sync_copy(oc_v, outc_hbm.at[pl.ds(row0 * 16, rpw * 16)])

    return sc_kernel


def kernel(feat_x, feat_y):
    B, Nx, C = feat_x.shape
    Ny = feat_y.shape[1]
    nchunks = -(-Ny // _CHUNK)
    nyp = nchunks * _CHUNK
    ngroups = nyp // _GRP
    nrows = B * Nx

    fyp = jnp.pad(feat_y, ((0, 0), (0, nyp - Ny), (0, 0)))

    sim, g = pl.pallas_call(
        functools.partial(_tc_body, Ny),
        grid=(B, nchunks),
        in_specs=[
            pl.BlockSpec((1, Nx, C), lambda b, c: (b, 0, 0)),
            pl.BlockSpec((1, _CHUNK, C), lambda b, c: (b, c, 0)),
        ],
        out_specs=[
            pl.BlockSpec((1, Nx, _CHUNK), lambda b, c: (b, 0, c)),
            pl.BlockSpec((1, 1, Nx, _GPC), lambda b, c: (b, c, 0, 0)),
        ],
        out_shape=[
            jax.ShapeDtypeStruct((B, Nx, nyp), jnp.float32),
            jax.ShapeDtypeStruct((B, nchunks, Nx, _GPC), jnp.float32),
        ],
    )(feat_x, fyp)

    sim_slabs = sim.reshape(nrows * ngroups, _GRP)
    g_flat = g.reshape(-1)
    outv, outc = _make_sc_kernel(nrows, nchunks, ngroups)(sim_slabs, g_flat)

    values = outv.reshape(nrows, 16)[:, :_K].reshape(-1)
    cols = outc.reshape(nrows, 16)[:, :_K].reshape(-1)
    bcol = jnp.repeat(jnp.arange(B, dtype=jnp.int32), Nx * _K)
    rows = jnp.tile(jnp.repeat(jnp.arange(Nx, dtype=jnp.int32), _K), B)
    indices = jnp.stack([bcol, rows, cols], axis=0)
    return indices, values
